# Initial kernel scaffold; baseline (speedup 1.0000x reference)
import jax
import jax.numpy as jnp
from jax.experimental import pallas as pl


def kernel(intervals, embed_weight):
    raise NotImplementedError("write your pallas kernel here")


# speedup vs baseline: 4.1162x; 4.1162x over previous
import functools

import jax
import jax.numpy as jnp
from jax import lax
from jax.experimental import pallas as pl
from jax.experimental.pallas import tpu as pltpu
from jax.experimental.pallas import tpu_sc as plsc

_NUM_BINS = 1000
_D = 64
_BATCH = 16384
_HIST = 200
_TOTAL = _BATCH * _HIST

_NC = 2
_NS = 16
_NW = _NC * _NS
_BW = _TOTAL // _NW
_C = 512
_NJ = _C // 128
_NB = _BW // _C


def _body(iv_hbm, tab_hbm, out_hbm, iv, idx, rows, gsem0, gsem1, osem0, osem1):
    wid = lax.axis_index("s") * _NC + lax.axis_index("c")
    base0 = wid * _BW
    gsems = (gsem0, gsem1)
    osems = (osem0, osem1)

    def stage(g, b):
        start = base0 + g * _C
        pltpu.sync_copy(iv_hbm.at[pl.ds(start, _C)], iv.at[b])
        iv_b = iv.at[b]
        for j in range(_NJ):
            idx_bj = idx.at[b].at[j]
            for i in range(128 // 16):
                v = iv_b[pl.ds(j * 128 + i * 16, 16)]
                bins = jnp.minimum(
                    lax.shift_right_logical(v * 37450, 18), _NUM_BINS - 1
                )
                idx_bj[pl.ds(i * 16, 16)] = bins
        for j in range(_NJ):
            pltpu.async_copy(
                tab_hbm.at[idx.at[b].at[j]],
                rows.at[b].at[pl.ds(j * 128, 128)],
                gsems[b],
            )

    def drain_gathers(b):
        for j in range(_NJ):
            pltpu.make_async_copy(
                tab_hbm.at[idx.at[b].at[j]],
                rows.at[b].at[pl.ds(j * 128, 128)],
                gsems[b],
            ).wait()

    def fire_out(g, b):
        start = base0 + g * _C
        pltpu.async_copy(rows.at[b], out_hbm.at[pl.ds(start, _C)], osems[b])

    def drain_out(g, b):
        start = base0 + g * _C
        pltpu.make_async_copy(
            rows.at[b], out_hbm.at[pl.ds(start, _C)], osems[b]
        ).wait()

    stage(0, 0)
    stage(1, 1)

    def loop_body(i, carry):
        g = i * 2
        drain_gathers(0)
        fire_out(g, 0)
        drain_gathers(1)
        fire_out(g + 1, 1)

        @pl.when(g + 2 < _NB)
        def _():
            drain_out(g, 0)
            stage(g + 2, 0)
            drain_out(g + 1, 1)
            stage(g + 3, 1)

        return carry

    lax.fori_loop(0, _NB // 2, loop_body, 0)
    drain_out(_NB - 2, 0)
    drain_out(_NB - 1, 1)


_sc_lookup = functools.partial(
    pl.kernel,
    out_type=jax.ShapeDtypeStruct((_TOTAL, _D), jnp.float32),
    mesh=plsc.VectorSubcoreMesh(core_axis_name="c", subcore_axis_name="s"),
    compiler_params=pltpu.CompilerParams(use_tc_tiling_on_sc=False),
    scratch_types=[
        pltpu.VMEM((2, _C), jnp.int32),
        pltpu.VMEM((2, _NJ, 128), jnp.int32),
        pltpu.VMEM((2, _C, _D), jnp.float32),
        pltpu.SemaphoreType.DMA,
        pltpu.SemaphoreType.DMA,
        pltpu.SemaphoreType.DMA,
        pltpu.SemaphoreType.DMA,
    ],
)(_body)


@jax.jit
def kernel(intervals, embed_weight):
    out = _sc_lookup(intervals.reshape(_TOTAL), embed_weight)
    return out.reshape(_BATCH, _HIST, _D)
